# baseline (speedup 1.0000x reference)
import jax
import jax.numpy as jnp
from jax.experimental import pallas as pl
from jax.experimental.pallas import tpu as pltpu

EPS = 1e-5
R = 128
WB = R + 8
U = 4
P = 2

BF = jnp.bfloat16


def _band_onehot(starts_ref, ends_ref, s0a, r0):
    sb = starts_ref[pl.ds(s0a, WB), :]
    eb = ends_ref[pl.ds(s0a, WB), :]
    row = jax.lax.broadcasted_iota(jnp.int32, (1, R), 1) + r0
    mask = (row >= sb) & (row < eb)
    return jnp.where(mask, 1.0, 0.0).astype(BF)


def _gather_rows(onehot, band_mat):
    return jax.lax.dot_general(
        onehot, band_mat, (((0,), (0,)), ((), ())),
        preferred_element_type=jnp.float32)


def _norm_rows(onehot, acc0, acc1, cnt, gamma_ref, beta_ref, y, d, chunk):
    outs = []
    for dc in range(0, d, chunk):
        sums = acc0[:, dc:dc + chunk] + acc1[:, dc:dc + chunk]
        sqs = acc0[:, d + dc:d + dc + chunk] + acc1[:, d + dc:d + dc + chunk]
        inv_cnt = 1.0 / jnp.maximum(cnt, 1.0)
        mean = sums * inv_cnt
        var = sqs * inv_cnt - mean * mean
        inv = jax.lax.rsqrt(var + EPS)
        a = inv * gamma_ref[0:1, dc:dc + chunk]
        c = beta_ref[0:1, dc:dc + chunk] - mean * a
        ac = jnp.concatenate([a, c], axis=1).astype(BF)
        rows = _gather_rows(onehot, ac)
        outs.append(y[:, dc:dc + chunk] * rows[:, :chunk]
                    + rows[:, chunk:])
    return jnp.concatenate(outs, axis=1)


def _gravity_y1(rel, h, mass_ref, wst_ref, bs_ref, w1t_ref, b1_ref, nk):
    cols = []
    for k in range(nk):
        x = rel[:, k:k + 1]
        y = rel[:, nk + k:nk + k + 1]
        inv_d = jax.lax.rsqrt(x * x + y * y)
        f = mass_ref[0, k] * (inv_d * inv_d)
        cols.append(-x * f)
        cols.append(-y * f)
    rep = jnp.concatenate(cols, axis=1)
    emb = jnp.dot(rep, wst_ref[...], preferred_element_type=jnp.float32)
    emb = emb + bs_ref[...]
    xcat = jnp.concatenate([emb, h], axis=1).astype(BF)
    y1 = jnp.dot(xcat, w1t_ref[...], preferred_element_type=jnp.float32)
    return y1 + b1_ref[...]


def kernel(h_state, seq_start_end, curr_block_rel, biker_mass, obstacle_mass,
           Ws, bs, W1, b1, g1, be1, W2, b2, g2, be2):
    n, h_dim = h_state.shape
    nk = curr_block_rel.shape[2]
    s = seq_start_end.shape[0]
    mid = W1.shape[0]
    bot = W2.shape[0]
    sp = s + 2 * WB
    nb = n // R
    nsteps = nb // (P * U)
    rb = U * R

    f32 = jnp.float32
    rel2 = curr_block_rel.reshape(n, 2 * nk).astype(f32)
    mass = (biker_mass[0] * obstacle_mass).reshape(1, nk).astype(f32)
    wst = Ws.T
    w1t = W1.T.astype(BF)
    w2t = W2.T.astype(BF)
    bs2 = bs.reshape(1, -1)
    b1r = b1.reshape(1, mid)
    g1r = g1.reshape(1, mid)
    be1r = be1.reshape(1, mid)
    b2r = b2.reshape(1, bot)
    g2r = g2.reshape(1, bot)
    be2r = be2.reshape(1, bot)

    starts = seq_start_end[:, 0].astype(jnp.int32)
    ends = seq_start_end[:, 1].astype(jnp.int32)
    padv = jnp.full((sp - s,), n, dtype=jnp.int32)
    starts_rep = jnp.broadcast_to(
        jnp.concatenate([starts, padv])[:, None], (sp, R))
    ends_rep = jnp.broadcast_to(
        jnp.concatenate([ends, padv])[:, None], (sp, R))
    blk0 = jnp.arange(nb, dtype=jnp.int32) * R
    s0a = ((jnp.searchsorted(ends, blk0, side="right").astype(jnp.int32)
            // 8) * 8)
    s1 = jnp.searchsorted(ends, blk0 + (R - 1), side="right").astype(jnp.int32)
    span = s1 - s0a + 1
    plan = jnp.concatenate([s0a, span])

    row_spec = lambda d: pl.BlockSpec(
        (rb, d), lambda p, j, sr: (p * nsteps + j, 0))
    const_spec = lambda shape: pl.BlockSpec(
        shape, lambda p, j, sr: tuple(0 for _ in shape))
    acc_spec = lambda d: pl.BlockSpec((1, sp, d), lambda p, j, sr: (p, 0, 0))

    nsteps_c = nsteps
    nb_c = nb

    def _sub(sr, pgid, j):
        g0 = (pgid * nsteps_c + j) * U
        return [(pl.multiple_of(sr[g0 + i], 8), sr[nb_c + g0 + i],
                 (g0 + i) * R) for i in range(U)]

    def _scatter_acc(acc_r, s0, span, part):
        for c in range(0, WB, 32):
            w = min(32, WB - c)

            @pl.when(c < span)
            def _():
                acc_r[0, pl.ds(s0 + c, w), :] += part[c:c + w, :]

    def _band_cnt(st_r, en_r, s0):
        sb = st_r[pl.ds(s0, WB), 0:1]
        eb = en_r[pl.ds(s0, WB), 0:1]
        return (eb - sb).astype(jnp.float32)

    def pass_a(sr, rel_r, h_r, mass_r, wst_r, bs_r, w1t_r, b1_r,
               st_r, en_r, acc1_r):
        subs = _sub(sr, pl.program_id(0), pl.program_id(1))

        @pl.when(pl.program_id(1) == 0)
        def _():
            acc1_r[...] = jnp.zeros_like(acc1_r)

        for i, (s0, span, r0) in enumerate(subs):
            sl = slice(i * R, (i + 1) * R)
            y1 = _gravity_y1(rel_r[sl, :], h_r[sl, :], mass_r, wst_r, bs_r,
                             w1t_r, b1_r, nk)
            z = jnp.concatenate([y1, y1 * y1], axis=1).astype(BF)
            oh = _band_onehot(st_r, en_r, s0, r0)
            part = jnp.dot(oh, z, preferred_element_type=jnp.float32)
            _scatter_acc(acc1_r, s0, span, part)

    acc1 = pl.pallas_call(
        pass_a,
        grid_spec=pltpu.PrefetchScalarGridSpec(
            num_scalar_prefetch=1,
            grid=(P, nsteps),
            in_specs=[
                row_spec(2 * nk), row_spec(h_dim), const_spec((1, nk)),
                const_spec((2 * nk, 16 * nk)), const_spec((1, 16 * nk)),
                const_spec((16 * nk + h_dim, mid)), const_spec((1, mid)),
                const_spec((sp, R)), const_spec((sp, R)),
            ],
            out_specs=acc_spec(2 * mid),
        ),
        out_shape=jax.ShapeDtypeStruct((P, sp, 2 * mid), f32),
        compiler_params=pltpu.CompilerParams(
            dimension_semantics=("parallel", "arbitrary"),
            vmem_limit_bytes=40 * 1024 * 1024,
        ),
        name="gravity_stats1",
    )(plan, rel2, h_state, mass, wst, bs2, w1t, b1r, starts_rep, ends_rep)

    def pass_b(sr, rel_r, h_r, mass_r, wst_r, bs_r, w1t_r, b1_r, g1_r, be1_r,
               w2t_r, b2_r, st_r, en_r, acc1_r, y2_r, acc2_r):
        subs = _sub(sr, pl.program_id(0), pl.program_id(1))

        @pl.when(pl.program_id(1) == 0)
        def _():
            acc2_r[...] = jnp.zeros_like(acc2_r)

        for i, (s0, span, r0) in enumerate(subs):
            sl = slice(i * R, (i + 1) * R)
            y1 = _gravity_y1(rel_r[sl, :], h_r[sl, :], mass_r, wst_r, bs_r,
                             w1t_r, b1_r, nk)
            oh = _band_onehot(st_r, en_r, s0, r0)
            cnt = _band_cnt(st_r, en_r, s0)
            acc0 = acc1_r[0, pl.ds(s0, WB), :]
            acc1b = acc1_r[1, pl.ds(s0, WB), :]
            h1 = jnp.maximum(
                _norm_rows(oh, acc0, acc1b, cnt, g1_r, be1_r, y1, mid, mid),
                0.0).astype(BF)
            y2 = jnp.dot(h1, w2t_r[...], preferred_element_type=jnp.float32)
            y2 = y2 + b2_r[...]
            y2_r[sl, :] = y2
            z2 = jnp.concatenate([y2, y2 * y2], axis=1).astype(BF)
            part2 = jnp.dot(oh, z2, preferred_element_type=jnp.float32)
            _scatter_acc(acc2_r, s0, span, part2)

    y2_full, acc2 = pl.pallas_call(
        pass_b,
        grid_spec=pltpu.PrefetchScalarGridSpec(
            num_scalar_prefetch=1,
            grid=(P, nsteps),
            in_specs=[
                row_spec(2 * nk), row_spec(h_dim), const_spec((1, nk)),
                const_spec((2 * nk, 16 * nk)), const_spec((1, 16 * nk)),
                const_spec((16 * nk + h_dim, mid)), const_spec((1, mid)),
                const_spec((1, mid)), const_spec((1, mid)),
                const_spec((mid, bot)), const_spec((1, bot)),
                const_spec((sp, R)), const_spec((sp, R)),
                const_spec((P, sp, 2 * mid)),
            ],
            out_specs=[row_spec(bot), acc_spec(2 * bot)],
        ),
        out_shape=[
            jax.ShapeDtypeStruct((n, bot), f32),
            jax.ShapeDtypeStruct((P, sp, 2 * bot), f32),
        ],
        compiler_params=pltpu.CompilerParams(
            dimension_semantics=("parallel", "arbitrary"),
            vmem_limit_bytes=52 * 1024 * 1024,
        ),
        name="gravity_mid",
    )(plan, rel2, h_state, mass, wst, bs2, w1t, b1r, g1r, be1r, w2t, b2r,
      starts_rep, ends_rep, acc1)

    def pass_c(sr, y2_r, g2_r, be2_r, st_r, en_r, acc2_r, out_r):
        subs = _sub(sr, pl.program_id(0), pl.program_id(1))
        for i, (s0, span, r0) in enumerate(subs):
            sl = slice(i * R, (i + 1) * R)
            oh = _band_onehot(st_r, en_r, s0, r0)
            cnt = _band_cnt(st_r, en_r, s0)
            acc0 = acc2_r[0, pl.ds(s0, WB), :]
            acc1b = acc2_r[1, pl.ds(s0, WB), :]
            y2 = y2_r[sl, :]
            out_r[sl, :] = jnp.maximum(
                _norm_rows(oh, acc0, acc1b, cnt, g2_r, be2_r, y2, bot, 512),
                0.0)

    out = pl.pallas_call(
        pass_c,
        grid_spec=pltpu.PrefetchScalarGridSpec(
            num_scalar_prefetch=1,
            grid=(P, nsteps),
            in_specs=[
                row_spec(bot), const_spec((1, bot)), const_spec((1, bot)),
                const_spec((sp, R)), const_spec((sp, R)),
                const_spec((P, sp, 2 * bot)),
            ],
            out_specs=row_spec(bot),
        ),
        out_shape=jax.ShapeDtypeStruct((n, bot), f32),
        compiler_params=pltpu.CompilerParams(
            dimension_semantics=("parallel", "arbitrary"),
            vmem_limit_bytes=52 * 1024 * 1024,
        ),
        name="gravity_norm2",
    )(plan, y2_full, g2r, be2r, starts_rep, ends_rep, acc2)

    return out


# speedup vs baseline: 1.3010x; 1.3010x over previous
import jax
import jax.numpy as jnp
from jax.experimental import pallas as pl
from jax.experimental.pallas import tpu as pltpu

EPS = 1e-5
R = 128
WB = R + 8
U = 4
P = 2

BF = jnp.bfloat16


def _band_onehot(starts_ref, ends_ref, s0a, r0):
    sb = starts_ref[pl.ds(s0a, WB), :]
    eb = ends_ref[pl.ds(s0a, WB), :]
    row = jax.lax.broadcasted_iota(jnp.int32, (1, R), 1) + r0
    mask = (row >= sb) & (row < eb)
    return jnp.where(mask, 1.0, 0.0).astype(BF)


def _band_affine(acc_band, cnt, gamma, beta, d):
    inv_cnt = 1.0 / jnp.maximum(cnt, 1.0)
    mean = acc_band[:, :d] * inv_cnt
    var = acc_band[:, d:] * inv_cnt - mean * mean
    inv = jax.lax.rsqrt(var + EPS)
    a = inv * gamma
    c = beta - mean * a
    return jnp.concatenate([a, c], axis=1)


def _gather_rows(onehot, band_mat):
    return jax.lax.dot_general(
        onehot, band_mat.astype(BF), (((0,), (0,)), ((), ())),
        preferred_element_type=jnp.float32)


def _gravity_y1(rel_ref, h_ref, mass_ref, wst_ref, bs_ref, w1t_ref, b1_ref, nk):
    rel = rel_ref[...]
    cols = []
    for k in range(nk):
        x = rel[:, k:k + 1]
        y = rel[:, nk + k:nk + k + 1]
        inv_d = jax.lax.rsqrt(x * x + y * y)
        f = mass_ref[0, k] * (inv_d * inv_d)
        cols.append(-x * f)
        cols.append(-y * f)
    rep = jnp.concatenate(cols, axis=1)
    emb = jnp.dot(rep, wst_ref[...], preferred_element_type=jnp.float32)
    emb = emb + bs_ref[...]
    xcat = jnp.concatenate([emb, h_ref[...]], axis=1).astype(BF)
    y1 = jnp.dot(xcat, w1t_ref[...], preferred_element_type=jnp.float32)
    return y1 + b1_ref[...]


def kernel(h_state, seq_start_end, curr_block_rel, biker_mass, obstacle_mass,
           Ws, bs, W1, b1, g1, be1, W2, b2, g2, be2):
    n, h_dim = h_state.shape
    nk = curr_block_rel.shape[2]
    s = seq_start_end.shape[0]
    mid = W1.shape[0]
    bot = W2.shape[0]
    sp = s + 2 * WB
    nb = n // R
    nsteps = nb // (P * U)
    rb = U * R

    f32 = jnp.float32
    rel2 = curr_block_rel.reshape(n, 2 * nk).astype(f32)
    mass = (biker_mass[0] * obstacle_mass).reshape(1, nk).astype(f32)
    wst = Ws.T
    w1t = W1.T.astype(BF)
    w2t = W2.T.astype(BF)
    bs2 = bs.reshape(1, -1)
    b1r = b1.reshape(1, mid)
    g1r = g1.reshape(1, mid)
    be1r = be1.reshape(1, mid)
    b2r = b2.reshape(1, bot)
    g2r = g2.reshape(1, bot)
    be2r = be2.reshape(1, bot)

    starts = seq_start_end[:, 0].astype(jnp.int32)
    ends = seq_start_end[:, 1].astype(jnp.int32)
    padv = jnp.full((sp - s,), n, dtype=jnp.int32)
    starts_rep = jnp.broadcast_to(
        jnp.concatenate([starts, padv])[:, None], (sp, R))
    ends_rep = jnp.broadcast_to(
        jnp.concatenate([ends, padv])[:, None], (sp, R))
    blk0 = jnp.arange(nb, dtype=jnp.int32) * R
    s0a = ((jnp.searchsorted(ends, blk0, side="right").astype(jnp.int32)
            // 8) * 8)
    s1 = jnp.searchsorted(ends, blk0 + (R - 1), side="right").astype(jnp.int32)
    span = s1 - s0a + 1
    plan = jnp.concatenate([s0a, span])

    row_spec = lambda d: pl.BlockSpec(
        (rb, d), lambda p, j, sr: (p * nsteps + j, 0))
    const_spec = lambda shape: pl.BlockSpec(
        shape, lambda p, j, sr: tuple(0 for _ in shape))
    acc_spec = lambda d: pl.BlockSpec((1, sp, d), lambda p, j, sr: (p, 0, 0))

    nsteps_c = nsteps

    nb_c = nb

    def _sub(sr, pgid, j):
        g0 = (pgid * nsteps_c + j) * U
        return [(pl.multiple_of(sr[g0 + i], 8), sr[nb_c + g0 + i],
                 (g0 + i) * R) for i in range(U)]

    def _scatter_acc(acc_r, s0, span, part, d):
        for c in range(0, WB, 32):
            w = min(32, WB - c)

            @pl.when(c < span)
            def _():
                acc_r[0, pl.ds(s0 + c, w), :] += part[c:c + w, :]

    def pass_a(sr, rel_r, h_r, mass_r, wst_r, bs_r, w1t_r, b1_r,
               st_r, en_r, acc1_r):
        subs = _sub(sr, pl.program_id(0), pl.program_id(1))
        y1 = _gravity_y1(rel_r, h_r, mass_r, wst_r, bs_r, w1t_r, b1_r, nk)
        z = jnp.concatenate([y1, y1 * y1], axis=1).astype(BF)

        @pl.when(pl.program_id(1) == 0)
        def _():
            acc1_r[...] = jnp.zeros_like(acc1_r)

        for i, (s0, span, r0) in enumerate(subs):
            oh = _band_onehot(st_r, en_r, s0, r0)
            part = jnp.dot(oh, z[i * R:(i + 1) * R, :],
                           preferred_element_type=jnp.float32)
            _scatter_acc(acc1_r, s0, span, part, 2 * mid)

    acc1 = pl.pallas_call(
        pass_a,
        grid_spec=pltpu.PrefetchScalarGridSpec(
            num_scalar_prefetch=1,
            grid=(P, nsteps),
            in_specs=[
                row_spec(2 * nk), row_spec(h_dim), const_spec((1, nk)),
                const_spec((2 * nk, 16 * nk)), const_spec((1, 16 * nk)),
                const_spec((16 * nk + h_dim, mid)), const_spec((1, mid)),
                const_spec((sp, R)), const_spec((sp, R)),
            ],
            out_specs=acc_spec(2 * mid),
        ),
        out_shape=jax.ShapeDtypeStruct((P, sp, 2 * mid), f32),
        compiler_params=pltpu.CompilerParams(
            dimension_semantics=("parallel", "arbitrary"),
            vmem_limit_bytes=40 * 1024 * 1024,
        ),
        name="gravity_stats1",
    )(plan, rel2, h_state, mass, wst, bs2, w1t, b1r, starts_rep, ends_rep)

    def pass_b1(sr, rel_r, h_r, mass_r, wst_r, bs_r, w1t_r, b1_r, g1_r, be1_r,
                w2t_r, b2_r, st_r, en_r, acc1_r, y2_r):
        subs = _sub(sr, pl.program_id(0), pl.program_id(1))
        y1 = _gravity_y1(rel_r, h_r, mass_r, wst_r, bs_r, w1t_r, b1_r, nk)
        d1 = y1.shape[1]

        h1_parts = []
        for i, (s0, span, r0) in enumerate(subs):
            oh = _band_onehot(st_r, en_r, s0, r0)
            band = acc1_r[0, pl.ds(s0, WB), :] + acc1_r[1, pl.ds(s0, WB), :]
            sb = st_r[pl.ds(s0, WB), 0:1]
            eb = en_r[pl.ds(s0, WB), 0:1]
            cnt = (eb - sb).astype(jnp.float32)
            ac = _band_affine(band, cnt, g1_r[...], be1_r[...], d1)
            rows = _gather_rows(oh, ac)
            y1_i = y1[i * R:(i + 1) * R, :]
            h1_parts.append(
                jnp.maximum(y1_i * rows[:, :d1] + rows[:, d1:], 0.0)
                .astype(BF))
        h1 = jnp.concatenate(h1_parts, axis=0)

        y2 = jnp.dot(h1, w2t_r[...], preferred_element_type=jnp.float32)
        y2_r[...] = y2 + b2_r[...]

    y2_full = pl.pallas_call(
        pass_b1,
        grid_spec=pltpu.PrefetchScalarGridSpec(
            num_scalar_prefetch=1,
            grid=(P, nsteps),
            in_specs=[
                row_spec(2 * nk), row_spec(h_dim), const_spec((1, nk)),
                const_spec((2 * nk, 16 * nk)), const_spec((1, 16 * nk)),
                const_spec((16 * nk + h_dim, mid)), const_spec((1, mid)),
                const_spec((1, mid)), const_spec((1, mid)),
                const_spec((mid, bot)), const_spec((1, bot)),
                const_spec((sp, R)), const_spec((sp, R)),
                const_spec((P, sp, 2 * mid)),
            ],
            out_specs=row_spec(bot),
        ),
        out_shape=jax.ShapeDtypeStruct((n, bot), f32),
        compiler_params=pltpu.CompilerParams(
            dimension_semantics=("parallel", "arbitrary"),
            vmem_limit_bytes=52 * 1024 * 1024,
        ),
        name="gravity_mid",
    )(plan, rel2, h_state, mass, wst, bs2, w1t, b1r, g1r, be1r, w2t, b2r,
      starts_rep, ends_rep, acc1)

    def pass_b2(sr, y2_r, st_r, en_r, acc2_r):
        subs = _sub(sr, pl.program_id(0), pl.program_id(1))

        @pl.when(pl.program_id(1) == 0)
        def _():
            acc2_r[...] = jnp.zeros_like(acc2_r)

        for i, (s0, span, r0) in enumerate(subs):
            y2_i = y2_r[i * R:(i + 1) * R, :]
            z2 = jnp.concatenate([y2_i, y2_i * y2_i], axis=1).astype(BF)
            oh = _band_onehot(st_r, en_r, s0, r0)
            part2 = jnp.dot(oh, z2, preferred_element_type=jnp.float32)
            _scatter_acc(acc2_r, s0, span, part2, 2 * bot)

    acc2 = pl.pallas_call(
        pass_b2,
        grid_spec=pltpu.PrefetchScalarGridSpec(
            num_scalar_prefetch=1,
            grid=(P, nsteps),
            in_specs=[
                row_spec(bot), const_spec((sp, R)), const_spec((sp, R)),
            ],
            out_specs=acc_spec(2 * bot),
        ),
        out_shape=jax.ShapeDtypeStruct((P, sp, 2 * bot), f32),
        compiler_params=pltpu.CompilerParams(
            dimension_semantics=("parallel", "arbitrary"),
            vmem_limit_bytes=52 * 1024 * 1024,
        ),
        name="gravity_stats2",
    )(plan, y2_full, starts_rep, ends_rep)

    def pass_c(sr, y2_r, g2_r, be2_r, st_r, en_r, acc2_r, out_r):
        subs = _sub(sr, pl.program_id(0), pl.program_id(1))
        for i, (s0, span, r0) in enumerate(subs):
            oh = _band_onehot(st_r, en_r, s0, r0)
            band = acc2_r[0, pl.ds(s0, WB), :] + acc2_r[1, pl.ds(s0, WB), :]
            sb = st_r[pl.ds(s0, WB), 0:1]
            eb = en_r[pl.ds(s0, WB), 0:1]
            cnt = (eb - sb).astype(jnp.float32)
            y2 = y2_r[i * R:(i + 1) * R, :]
            d2 = y2.shape[1]
            ac = _band_affine(band, cnt, g2_r[...], be2_r[...], d2)
            rows = _gather_rows(oh, ac)
            out_r[i * R:(i + 1) * R, :] = jnp.maximum(
                y2 * rows[:, :d2] + rows[:, d2:], 0.0)

    out = pl.pallas_call(
        pass_c,
        grid_spec=pltpu.PrefetchScalarGridSpec(
            num_scalar_prefetch=1,
            grid=(P, nsteps),
            in_specs=[
                row_spec(bot), const_spec((1, bot)), const_spec((1, bot)),
                const_spec((sp, R)), const_spec((sp, R)),
                const_spec((P, sp, 2 * bot)),
            ],
            out_specs=row_spec(bot),
        ),
        out_shape=jax.ShapeDtypeStruct((n, bot), f32),
        compiler_params=pltpu.CompilerParams(
            dimension_semantics=("parallel", "arbitrary"),
            vmem_limit_bytes=52 * 1024 * 1024,
        ),
        name="gravity_norm2",
    )(plan, y2_full, g2r, be2r, starts_rep, ends_rep, acc2)

    return out


# speedup vs baseline: 1.6168x; 1.2428x over previous
import jax
import jax.numpy as jnp
from jax.experimental import pallas as pl
from jax.experimental.pallas import tpu as pltpu

EPS = 1e-5
R = 128
WB = R + 8
U = 4
P = 2

BF = jnp.bfloat16


def _band_onehot(starts_ref, ends_ref, s0a, r0):
    sb = starts_ref[pl.ds(s0a, WB), :]
    eb = ends_ref[pl.ds(s0a, WB), :]
    row = jax.lax.broadcasted_iota(jnp.int32, (1, R), 1) + r0
    mask = (row >= sb) & (row < eb)
    return jnp.where(mask, 1.0, 0.0).astype(BF)


def _band_affine(acc_band, cnt, gamma, beta, d):
    inv_cnt = 1.0 / jnp.maximum(cnt, 1.0)
    mean = acc_band[:, :d] * inv_cnt
    var = acc_band[:, d:] * inv_cnt - mean * mean
    inv = jax.lax.rsqrt(var + EPS)
    a = inv * gamma
    c = beta - mean * a
    return jnp.concatenate([a, c], axis=1)


def _gather_rows(onehot, band_mat):
    return jax.lax.dot_general(
        onehot, band_mat, (((0,), (0,)), ((), ())),
        preferred_element_type=jnp.float32)


def _gravity_y1(rel_ref, h_ref, mass_ref, wst_ref, bs_ref, w1t_ref, b1_ref, nk):
    rel = rel_ref[...]
    cols = []
    for k in range(nk):
        x = rel[:, k:k + 1]
        y = rel[:, nk + k:nk + k + 1]
        inv_d = jax.lax.rsqrt(x * x + y * y)
        f = mass_ref[0, k] * (inv_d * inv_d)
        cols.append(-x * f)
        cols.append(-y * f)
    rep = jnp.concatenate(cols, axis=1)
    emb = jnp.dot(rep, wst_ref[...], preferred_element_type=jnp.float32)
    emb = emb + bs_ref[...]
    xcat = jnp.concatenate([emb, h_ref[...]], axis=1).astype(BF)
    y1 = jnp.dot(xcat, w1t_ref[...], preferred_element_type=jnp.float32)
    return y1 + b1_ref[...]


def kernel(h_state, seq_start_end, curr_block_rel, biker_mass, obstacle_mass,
           Ws, bs, W1, b1, g1, be1, W2, b2, g2, be2):
    n, h_dim = h_state.shape
    nk = curr_block_rel.shape[2]
    s = seq_start_end.shape[0]
    mid = W1.shape[0]
    bot = W2.shape[0]
    sp = s + 2 * WB
    nb = n // R
    nsteps = nb // (P * U)
    rb = U * R

    f32 = jnp.float32
    rel2 = curr_block_rel.reshape(n, 2 * nk).astype(f32)
    mass = (biker_mass[0] * obstacle_mass).reshape(1, nk).astype(f32)
    wst = Ws.T
    w1t = W1.T.astype(BF)
    w2t = W2.T.astype(BF)
    bs2 = bs.reshape(1, -1)
    b1r = b1.reshape(1, mid)
    g1r = g1.reshape(1, mid)
    be1r = be1.reshape(1, mid)
    b2r = b2.reshape(1, bot)
    g2r = g2.reshape(1, bot)
    be2r = be2.reshape(1, bot)

    starts = seq_start_end[:, 0].astype(jnp.int32)
    ends = seq_start_end[:, 1].astype(jnp.int32)
    padv = jnp.full((sp - s,), n, dtype=jnp.int32)
    starts_rep = jnp.broadcast_to(
        jnp.concatenate([starts, padv])[:, None], (sp, R))
    ends_rep = jnp.broadcast_to(
        jnp.concatenate([ends, padv])[:, None], (sp, R))
    blk0 = jnp.arange(nb, dtype=jnp.int32) * R
    s0a = ((jnp.searchsorted(ends, blk0, side="right").astype(jnp.int32)
            // 8) * 8)
    s1 = jnp.searchsorted(ends, blk0 + (R - 1), side="right").astype(jnp.int32)
    span = s1 - s0a + 1
    plan = jnp.concatenate([s0a, span])

    row_spec = lambda d: pl.BlockSpec(
        (rb, d), lambda p, j, sr: (p * nsteps + j, 0))
    const_spec = lambda shape: pl.BlockSpec(
        shape, lambda p, j, sr: tuple(0 for _ in shape))
    acc_spec = lambda d: pl.BlockSpec((1, sp, d), lambda p, j, sr: (p, 0, 0))

    nsteps_c = nsteps

    nb_c = nb

    def _sub(sr, pgid, j):
        g0 = (pgid * nsteps_c + j) * U
        return [(pl.multiple_of(sr[g0 + i], 8), sr[nb_c + g0 + i],
                 (g0 + i) * R) for i in range(U)]

    def _scatter_acc(acc_r, s0, span, part, d):
        for c in range(0, WB, 32):
            w = min(32, WB - c)

            @pl.when(c < span)
            def _():
                acc_r[0, pl.ds(s0 + c, w), :] += part[c:c + w, :]

    def pass_a(sr, rel_r, h_r, mass_r, wst_r, bs_r, w1t_r, b1_r,
               st_r, en_r, acc1_r):
        subs = _sub(sr, pl.program_id(0), pl.program_id(1))
        y1b = _gravity_y1(rel_r, h_r, mass_r, wst_r, bs_r, w1t_r, b1_r,
                          nk).astype(BF)
        z = jnp.concatenate([y1b, y1b * y1b], axis=1)

        @pl.when(pl.program_id(1) == 0)
        def _():
            acc1_r[...] = jnp.zeros_like(acc1_r)

        for i, (s0, span, r0) in enumerate(subs):
            oh = _band_onehot(st_r, en_r, s0, r0)
            part = jnp.dot(oh, z[i * R:(i + 1) * R, :],
                           preferred_element_type=jnp.float32)
            _scatter_acc(acc1_r, s0, span, part, 2 * mid)

    acc1 = pl.pallas_call(
        pass_a,
        grid_spec=pltpu.PrefetchScalarGridSpec(
            num_scalar_prefetch=1,
            grid=(P, nsteps),
            in_specs=[
                row_spec(2 * nk), row_spec(h_dim), const_spec((1, nk)),
                const_spec((2 * nk, 16 * nk)), const_spec((1, 16 * nk)),
                const_spec((16 * nk + h_dim, mid)), const_spec((1, mid)),
                const_spec((sp, R)), const_spec((sp, R)),
            ],
            out_specs=acc_spec(2 * mid),
        ),
        out_shape=jax.ShapeDtypeStruct((P, sp, 2 * mid), f32),
        compiler_params=pltpu.CompilerParams(
            dimension_semantics=("parallel", "arbitrary"),
            vmem_limit_bytes=40 * 1024 * 1024,
        ),
        name="gravity_stats1",
    )(plan, rel2, h_state, mass, wst, bs2, w1t, b1r, starts_rep, ends_rep)

    AB = 128

    def _affine_call(acc, gam, bet, d, name):
        def aff_k(acc_r, st_r, en_r, g_r, be_r, af_r):
            sums = acc_r[0] + acc_r[1]
            cnt = (en_r[:, 0:1] - st_r[:, 0:1]).astype(jnp.float32)
            inv_cnt = 1.0 / jnp.maximum(cnt, 1.0)
            mean = sums[:, :d] * inv_cnt
            var = sums[:, d:] * inv_cnt - mean * mean
            inv = jax.lax.rsqrt(var + EPS)
            a = inv * g_r[...]
            c = be_r[...] - mean * a
            af_r[...] = jnp.concatenate([a, c], axis=1).astype(BF)

        return pl.pallas_call(
            aff_k,
            grid=(sp // AB,),
            in_specs=[
                pl.BlockSpec((P, AB, 2 * d), lambda j: (0, j, 0)),
                pl.BlockSpec((AB, R), lambda j: (j, 0)),
                pl.BlockSpec((AB, R), lambda j: (j, 0)),
                pl.BlockSpec((1, d), lambda j: (0, 0)),
                pl.BlockSpec((1, d), lambda j: (0, 0)),
            ],
            out_specs=pl.BlockSpec((AB, 2 * d), lambda j: (j, 0)),
            out_shape=jax.ShapeDtypeStruct((sp, 2 * d), BF),
            compiler_params=pltpu.CompilerParams(
                dimension_semantics=("arbitrary",)),
            name=name,
        )(acc, starts_rep, ends_rep, gam, bet)

    af1 = _affine_call(acc1, g1r, be1r, mid, "gravity_affine1")

    def pass_b(sr, rel_r, h_r, mass_r, wst_r, bs_r, w1t_r, b1_r,
               w2t_r, b2_r, st_r, en_r, af1_r, y2_r, acc2_r):
        subs = _sub(sr, pl.program_id(0), pl.program_id(1))
        y1 = _gravity_y1(rel_r, h_r, mass_r, wst_r, bs_r, w1t_r, b1_r, nk)
        d1 = y1.shape[1]

        ohs = []
        h1_parts = []
        for i, (s0, span, r0) in enumerate(subs):
            oh = _band_onehot(st_r, en_r, s0, r0)
            ohs.append(oh)
            rows = _gather_rows(oh, af1_r[pl.ds(s0, WB), :])
            y1_i = y1[i * R:(i + 1) * R, :]
            h1_parts.append(
                jnp.maximum(y1_i * rows[:, :d1] + rows[:, d1:], 0.0)
                .astype(BF))
        h1 = jnp.concatenate(h1_parts, axis=0)

        y2 = jnp.dot(h1, w2t_r[...], preferred_element_type=jnp.float32)
        y2 = y2 + b2_r[...]
        y2_r[...] = y2

        @pl.when(pl.program_id(1) == 0)
        def _():
            acc2_r[...] = jnp.zeros_like(acc2_r)

        for i, (s0, span, r0) in enumerate(subs):
            y2_i = y2[i * R:(i + 1) * R, :].astype(BF)
            z2 = jnp.concatenate([y2_i, y2_i * y2_i], axis=1)
            part2 = jnp.dot(ohs[i], z2, preferred_element_type=jnp.float32)
            _scatter_acc(acc2_r, s0, span, part2, 2 * bot)

    y2_full, acc2 = pl.pallas_call(
        pass_b,
        grid_spec=pltpu.PrefetchScalarGridSpec(
            num_scalar_prefetch=1,
            grid=(P, nsteps),
            in_specs=[
                row_spec(2 * nk), row_spec(h_dim), const_spec((1, nk)),
                const_spec((2 * nk, 16 * nk)), const_spec((1, 16 * nk)),
                const_spec((16 * nk + h_dim, mid)), const_spec((1, mid)),
                const_spec((mid, bot)), const_spec((1, bot)),
                const_spec((sp, R)), const_spec((sp, R)),
                const_spec((sp, 2 * mid)),
            ],
            out_specs=[row_spec(bot), acc_spec(2 * bot)],
        ),
        out_shape=[
            jax.ShapeDtypeStruct((n, bot), f32),
            jax.ShapeDtypeStruct((P, sp, 2 * bot), f32),
        ],
        compiler_params=pltpu.CompilerParams(
            dimension_semantics=("parallel", "arbitrary"),
            vmem_limit_bytes=52 * 1024 * 1024,
        ),
        name="gravity_mid",
    )(plan, rel2, h_state, mass, wst, bs2, w1t, b1r, w2t, b2r,
      starts_rep, ends_rep, af1)

    af2 = _affine_call(acc2, g2r, be2r, bot, "gravity_affine2")

    def pass_c(sr, y2_r, st_r, en_r, af2_r, out_r):
        subs = _sub(sr, pl.program_id(0), pl.program_id(1))
        for i, (s0, span, r0) in enumerate(subs):
            oh = _band_onehot(st_r, en_r, s0, r0)
            rows = _gather_rows(oh, af2_r[pl.ds(s0, WB), :])
            y2 = y2_r[i * R:(i + 1) * R, :]
            d2 = y2.shape[1]
            out_r[i * R:(i + 1) * R, :] = jnp.maximum(
                y2 * rows[:, :d2] + rows[:, d2:], 0.0)

    out = pl.pallas_call(
        pass_c,
        grid_spec=pltpu.PrefetchScalarGridSpec(
            num_scalar_prefetch=1,
            grid=(P, nsteps),
            in_specs=[
                row_spec(bot),
                const_spec((sp, R)), const_spec((sp, R)),
                const_spec((sp, 2 * bot)),
            ],
            out_specs=row_spec(bot),
        ),
        out_shape=jax.ShapeDtypeStruct((n, bot), f32),
        compiler_params=pltpu.CompilerParams(
            dimension_semantics=("parallel", "arbitrary"),
            vmem_limit_bytes=52 * 1024 * 1024,
        ),
        name="gravity_norm2",
    )(plan, y2_full, starts_rep, ends_rep, af2)

    return out


# speedup vs baseline: 1.6193x; 1.0015x over previous
import jax
import jax.numpy as jnp
from jax.experimental import pallas as pl
from jax.experimental.pallas import tpu as pltpu

EPS = 1e-5
R = 128
WB = R + 8
U = 4
P = 2

BF = jnp.bfloat16


def _band_onehot(starts_ref, ends_ref, s0a, r0):
    sb = starts_ref[pl.ds(s0a, WB), :]
    eb = ends_ref[pl.ds(s0a, WB), :]
    row = jax.lax.broadcasted_iota(jnp.int32, (1, R), 1) + r0
    mask = (row >= sb) & (row < eb)
    return jnp.where(mask, 1.0, 0.0).astype(BF)


def _band_affine(acc_band, cnt, gamma, beta, d):
    inv_cnt = 1.0 / jnp.maximum(cnt, 1.0)
    mean = acc_band[:, :d] * inv_cnt
    var = acc_band[:, d:] * inv_cnt - mean * mean
    inv = jax.lax.rsqrt(var + EPS)
    a = inv * gamma
    c = beta - mean * a
    return jnp.concatenate([a, c], axis=1)


def _gather_rows(onehot, band_mat):
    return jax.lax.dot_general(
        onehot, band_mat, (((0,), (0,)), ((), ())),
        preferred_element_type=jnp.float32)


def _gravity_y1(rel_ref, h_ref, mass_ref, wst_ref, bs_ref, w1t_ref, b1_ref, nk):
    rel_t = rel_ref[...]
    rows = []
    for k in range(nk):
        x = rel_t[k:k + 1, :]
        y = rel_t[nk + k:nk + k + 1, :]
        inv_d2 = 1.0 / (x * x + y * y)
        f = mass_ref[0, k] * inv_d2
        rows.append(-x * f)
        rows.append(-y * f)
    rep_t = jnp.concatenate(rows, axis=0)
    emb = jax.lax.dot_general(
        rep_t, wst_ref[...], (((0,), (0,)), ((), ())),
        preferred_element_type=jnp.float32)
    emb = emb + bs_ref[...]
    xcat = jnp.concatenate([emb, h_ref[...]], axis=1).astype(BF)
    y1 = jnp.dot(xcat, w1t_ref[...], preferred_element_type=jnp.float32)
    return y1 + b1_ref[...]


def kernel(h_state, seq_start_end, curr_block_rel, biker_mass, obstacle_mass,
           Ws, bs, W1, b1, g1, be1, W2, b2, g2, be2):
    n, h_dim = h_state.shape
    nk = curr_block_rel.shape[2]
    s = seq_start_end.shape[0]
    mid = W1.shape[0]
    bot = W2.shape[0]
    sp = s + 2 * WB
    nb = n // R
    nsteps = nb // (P * U)
    rb = U * R

    f32 = jnp.float32
    rel2t = curr_block_rel.reshape(n, 2 * nk).T.astype(f32)
    mass = (biker_mass[0] * obstacle_mass).reshape(1, nk).astype(f32)
    wst = Ws.T
    w1t = W1.T.astype(BF)
    w2t = W2.T.astype(BF)
    bs2 = bs.reshape(1, -1)
    b1r = b1.reshape(1, mid)
    g1r = g1.reshape(1, mid)
    be1r = be1.reshape(1, mid)
    b2r = b2.reshape(1, bot)
    g2r = g2.reshape(1, bot)
    be2r = be2.reshape(1, bot)

    starts = seq_start_end[:, 0].astype(jnp.int32)
    ends = seq_start_end[:, 1].astype(jnp.int32)
    padv = jnp.full((sp - s,), n, dtype=jnp.int32)
    starts_rep = jnp.broadcast_to(
        jnp.concatenate([starts, padv])[:, None], (sp, R))
    ends_rep = jnp.broadcast_to(
        jnp.concatenate([ends, padv])[:, None], (sp, R))
    blk0 = jnp.arange(nb, dtype=jnp.int32) * R
    s0a = ((jnp.searchsorted(ends, blk0, side="right").astype(jnp.int32)
            // 8) * 8)
    s1 = jnp.searchsorted(ends, blk0 + (R - 1), side="right").astype(jnp.int32)
    span = s1 - s0a + 1
    plan = jnp.concatenate([s0a, span])

    row_spec = lambda d: pl.BlockSpec(
        (rb, d), lambda p, j, sr: (p * nsteps + j, 0))
    relt_spec = pl.BlockSpec((2 * nk, rb), lambda p, j, sr: (0, p * nsteps + j))
    const_spec = lambda shape: pl.BlockSpec(
        shape, lambda p, j, sr: tuple(0 for _ in shape))
    acc_spec = lambda d: pl.BlockSpec((1, sp, d), lambda p, j, sr: (p, 0, 0))

    nsteps_c = nsteps

    nb_c = nb

    def _sub(sr, pgid, j):
        g0 = (pgid * nsteps_c + j) * U
        return [(pl.multiple_of(sr[g0 + i], 8), sr[nb_c + g0 + i],
                 (g0 + i) * R) for i in range(U)]

    def _scatter_acc(acc_r, s0, span, part, d):
        for c in range(0, WB, 32):
            w = min(32, WB - c)

            @pl.when(c < span)
            def _():
                acc_r[0, pl.ds(s0 + c, w), :] += part[c:c + w, :]

    def pass_a(sr, rel_r, h_r, mass_r, wst_r, bs_r, w1t_r, b1_r,
               st_r, en_r, acc1_r):
        subs = _sub(sr, pl.program_id(0), pl.program_id(1))
        y1b = _gravity_y1(rel_r, h_r, mass_r, wst_r, bs_r, w1t_r, b1_r,
                          nk).astype(BF)
        z = jnp.concatenate([y1b, y1b * y1b], axis=1)

        @pl.when(pl.program_id(1) == 0)
        def _():
            acc1_r[...] = jnp.zeros_like(acc1_r)

        for i, (s0, span, r0) in enumerate(subs):
            oh = _band_onehot(st_r, en_r, s0, r0)
            part = jnp.dot(oh, z[i * R:(i + 1) * R, :],
                           preferred_element_type=jnp.float32)
            _scatter_acc(acc1_r, s0, span, part, 2 * mid)

    acc1 = pl.pallas_call(
        pass_a,
        grid_spec=pltpu.PrefetchScalarGridSpec(
            num_scalar_prefetch=1,
            grid=(P, nsteps),
            in_specs=[
                relt_spec, row_spec(h_dim), const_spec((1, nk)),
                const_spec((2 * nk, 16 * nk)), const_spec((1, 16 * nk)),
                const_spec((16 * nk + h_dim, mid)), const_spec((1, mid)),
                const_spec((sp, R)), const_spec((sp, R)),
            ],
            out_specs=acc_spec(2 * mid),
        ),
        out_shape=jax.ShapeDtypeStruct((P, sp, 2 * mid), f32),
        compiler_params=pltpu.CompilerParams(
            dimension_semantics=("parallel", "arbitrary"),
            vmem_limit_bytes=40 * 1024 * 1024,
        ),
        name="gravity_stats1",
    )(plan, rel2t, h_state, mass, wst, bs2, w1t, b1r, starts_rep, ends_rep)

    AB = 128

    def _affine_call(acc, gam, bet, d, name):
        def aff_k(acc_r, st_r, en_r, g_r, be_r, af_r):
            sums = acc_r[0] + acc_r[1]
            cnt = (en_r[:, 0:1] - st_r[:, 0:1]).astype(jnp.float32)
            inv_cnt = 1.0 / jnp.maximum(cnt, 1.0)
            mean = sums[:, :d] * inv_cnt
            var = sums[:, d:] * inv_cnt - mean * mean
            inv = jax.lax.rsqrt(var + EPS)
            a = inv * g_r[...]
            c = be_r[...] - mean * a
            af_r[...] = jnp.concatenate([a, c], axis=1).astype(BF)

        return pl.pallas_call(
            aff_k,
            grid=(sp // AB,),
            in_specs=[
                pl.BlockSpec((P, AB, 2 * d), lambda j: (0, j, 0)),
                pl.BlockSpec((AB, R), lambda j: (j, 0)),
                pl.BlockSpec((AB, R), lambda j: (j, 0)),
                pl.BlockSpec((1, d), lambda j: (0, 0)),
                pl.BlockSpec((1, d), lambda j: (0, 0)),
            ],
            out_specs=pl.BlockSpec((AB, 2 * d), lambda j: (j, 0)),
            out_shape=jax.ShapeDtypeStruct((sp, 2 * d), BF),
            compiler_params=pltpu.CompilerParams(
                dimension_semantics=("arbitrary",)),
            name=name,
        )(acc, starts_rep, ends_rep, gam, bet)

    af1 = _affine_call(acc1, g1r, be1r, mid, "gravity_affine1")

    def pass_b(sr, rel_r, h_r, mass_r, wst_r, bs_r, w1t_r, b1_r,
               w2t_r, b2_r, st_r, en_r, af1_r, y2_r, acc2_r):
        subs = _sub(sr, pl.program_id(0), pl.program_id(1))
        y1 = _gravity_y1(rel_r, h_r, mass_r, wst_r, bs_r, w1t_r, b1_r, nk)
        d1 = y1.shape[1]

        ohs = []
        h1_parts = []
        for i, (s0, span, r0) in enumerate(subs):
            oh = _band_onehot(st_r, en_r, s0, r0)
            ohs.append(oh)
            rows = _gather_rows(oh, af1_r[pl.ds(s0, WB), :])
            y1_i = y1[i * R:(i + 1) * R, :]
            h1_parts.append(
                jnp.maximum(y1_i * rows[:, :d1] + rows[:, d1:], 0.0)
                .astype(BF))
        h1 = jnp.concatenate(h1_parts, axis=0)

        y2 = jnp.dot(h1, w2t_r[...], preferred_element_type=jnp.float32)
        y2 = (y2 + b2_r[...]).astype(BF)
        y2_r[...] = y2

        @pl.when(pl.program_id(1) == 0)
        def _():
            acc2_r[...] = jnp.zeros_like(acc2_r)

        for i, (s0, span, r0) in enumerate(subs):
            y2_i = y2[i * R:(i + 1) * R, :]
            z2 = jnp.concatenate([y2_i, y2_i * y2_i], axis=1)
            part2 = jnp.dot(ohs[i], z2, preferred_element_type=jnp.float32)
            _scatter_acc(acc2_r, s0, span, part2, 2 * bot)

    y2_full, acc2 = pl.pallas_call(
        pass_b,
        grid_spec=pltpu.PrefetchScalarGridSpec(
            num_scalar_prefetch=1,
            grid=(P, nsteps),
            in_specs=[
                relt_spec, row_spec(h_dim), const_spec((1, nk)),
                const_spec((2 * nk, 16 * nk)), const_spec((1, 16 * nk)),
                const_spec((16 * nk + h_dim, mid)), const_spec((1, mid)),
                const_spec((mid, bot)), const_spec((1, bot)),
                const_spec((sp, R)), const_spec((sp, R)),
                const_spec((sp, 2 * mid)),
            ],
            out_specs=[row_spec(bot), acc_spec(2 * bot)],
        ),
        out_shape=[
            jax.ShapeDtypeStruct((n, bot), BF),
            jax.ShapeDtypeStruct((P, sp, 2 * bot), f32),
        ],
        compiler_params=pltpu.CompilerParams(
            dimension_semantics=("parallel", "arbitrary"),
            vmem_limit_bytes=52 * 1024 * 1024,
        ),
        name="gravity_mid",
    )(plan, rel2t, h_state, mass, wst, bs2, w1t, b1r, w2t, b2r,
      starts_rep, ends_rep, af1)

    af2 = _affine_call(acc2, g2r, be2r, bot, "gravity_affine2")

    def pass_c(sr, y2_r, st_r, en_r, af2_r, out_r):
        subs = _sub(sr, pl.program_id(0), pl.program_id(1))
        for i, (s0, span, r0) in enumerate(subs):
            oh = _band_onehot(st_r, en_r, s0, r0)
            rows = _gather_rows(oh, af2_r[pl.ds(s0, WB), :])
            y2 = y2_r[i * R:(i + 1) * R, :].astype(jnp.float32)
            d2 = y2.shape[1]
            out_r[i * R:(i + 1) * R, :] = jnp.maximum(
                y2 * rows[:, :d2] + rows[:, d2:], 0.0)

    out = pl.pallas_call(
        pass_c,
        grid_spec=pltpu.PrefetchScalarGridSpec(
            num_scalar_prefetch=1,
            grid=(P, nsteps),
            in_specs=[
                row_spec(bot),
                const_spec((sp, R)), const_spec((sp, R)),
                const_spec((sp, 2 * bot)),
            ],
            out_specs=row_spec(bot),
        ),
        out_shape=jax.ShapeDtypeStruct((n, bot), f32),
        compiler_params=pltpu.CompilerParams(
            dimension_semantics=("parallel", "arbitrary"),
            vmem_limit_bytes=52 * 1024 * 1024,
        ),
        name="gravity_norm2",
    )(plan, y2_full, starts_rep, ends_rep, af2)

    return out


# speedup vs baseline: 1.8338x; 1.1325x over previous
import jax
import jax.numpy as jnp
from jax.experimental import pallas as pl
from jax.experimental.pallas import tpu as pltpu

EPS = 1e-5
R = 128
WB = R + 8
U = 8
P = 2

BF = jnp.bfloat16


def _band_onehot(starts_ref, ends_ref, s0a, r0):
    sb = starts_ref[pl.ds(s0a, WB), :]
    eb = ends_ref[pl.ds(s0a, WB), :]
    row = jax.lax.broadcasted_iota(jnp.int32, (1, R), 1) + r0
    mask = (row >= sb) & (row < eb)
    return jnp.where(mask, 1.0, 0.0).astype(BF)


def _band_affine(acc_band, cnt, gamma, beta, d):
    inv_cnt = 1.0 / jnp.maximum(cnt, 1.0)
    mean = acc_band[:, :d] * inv_cnt
    var = acc_band[:, d:] * inv_cnt - mean * mean
    inv = jax.lax.rsqrt(var + EPS)
    a = inv * gamma
    c = beta - mean * a
    return jnp.concatenate([a, c], axis=1)


def _gather_rows(onehot, band_mat):
    return jax.lax.dot_general(
        onehot, band_mat, (((0,), (0,)), ((), ())),
        preferred_element_type=jnp.float32)


def _gravity_y1(rel_ref, h_ref, mass_ref, wst_ref, bs_ref, w1t_ref, b1_ref, nk):
    rel_t = rel_ref[...]
    rows = []
    for k in range(nk):
        x = rel_t[k:k + 1, :]
        y = rel_t[nk + k:nk + k + 1, :]
        inv_d2 = 1.0 / (x * x + y * y)
        f = mass_ref[0, k] * inv_d2
        rows.append(-x * f)
        rows.append(-y * f)
    rep_t = jnp.concatenate(rows, axis=0)
    emb = jax.lax.dot_general(
        rep_t, wst_ref[...], (((0,), (0,)), ((), ())),
        preferred_element_type=jnp.float32)
    emb = emb + bs_ref[...]
    xcat = jnp.concatenate([emb, h_ref[...]], axis=1).astype(BF)
    y1 = jnp.dot(xcat, w1t_ref[...], preferred_element_type=jnp.float32)
    return y1 + b1_ref[...]


def kernel(h_state, seq_start_end, curr_block_rel, biker_mass, obstacle_mass,
           Ws, bs, W1, b1, g1, be1, W2, b2, g2, be2):
    n, h_dim = h_state.shape
    nk = curr_block_rel.shape[2]
    s = seq_start_end.shape[0]
    mid = W1.shape[0]
    bot = W2.shape[0]
    sp = s + 2 * WB
    nb = n // R
    nsteps = nb // (P * U)
    rb = U * R

    f32 = jnp.float32
    rel2t = curr_block_rel.reshape(n, 2 * nk).T.astype(f32)
    mass = (biker_mass[0] * obstacle_mass).reshape(1, nk).astype(f32)
    wst = Ws.T
    w1t = W1.T.astype(BF)
    w2t = W2.T.astype(BF)
    bs2 = bs.reshape(1, -1)
    b1r = b1.reshape(1, mid)
    g1r = g1.reshape(1, mid)
    be1r = be1.reshape(1, mid)
    b2r = b2.reshape(1, bot)
    g2r = g2.reshape(1, bot)
    be2r = be2.reshape(1, bot)

    starts = seq_start_end[:, 0].astype(jnp.int32)
    ends = seq_start_end[:, 1].astype(jnp.int32)
    padv = jnp.full((sp - s,), n, dtype=jnp.int32)
    starts_rep = jnp.broadcast_to(
        jnp.concatenate([starts, padv])[:, None], (sp, R))
    ends_rep = jnp.broadcast_to(
        jnp.concatenate([ends, padv])[:, None], (sp, R))
    blk0 = jnp.arange(nb, dtype=jnp.int32) * R
    s0a = ((jnp.searchsorted(ends, blk0, side="right").astype(jnp.int32)
            // 8) * 8)
    s1 = jnp.searchsorted(ends, blk0 + (R - 1), side="right").astype(jnp.int32)
    span = s1 - s0a + 1
    plan = jnp.concatenate([s0a, span])

    row_spec = lambda d: pl.BlockSpec(
        (rb, d), lambda p, j, sr: (p * nsteps + j, 0))
    relt_spec = pl.BlockSpec((2 * nk, rb), lambda p, j, sr: (0, p * nsteps + j))
    const_spec = lambda shape: pl.BlockSpec(
        shape, lambda p, j, sr: tuple(0 for _ in shape))
    acc_spec = lambda d: pl.BlockSpec((1, sp, d), lambda p, j, sr: (p, 0, 0))

    nsteps_c = nsteps

    nb_c = nb

    def _sub(sr, pgid, j):
        g0 = (pgid * nsteps_c + j) * U
        return [(pl.multiple_of(sr[g0 + i], 8), sr[nb_c + g0 + i],
                 (g0 + i) * R) for i in range(U)]

    def _scatter_acc(acc_r, s0, span, part, d):
        for c in range(0, WB, 32):
            w = min(32, WB - c)

            @pl.when(c < span)
            def _():
                acc_r[0, pl.ds(s0 + c, w), :] += part[c:c + w, :]

    def pass_a(sr, rel_r, h_r, mass_r, wst_r, bs_r, w1t_r, b1_r,
               st_r, en_r, acc1_r):
        subs = _sub(sr, pl.program_id(0), pl.program_id(1))
        y1b = _gravity_y1(rel_r, h_r, mass_r, wst_r, bs_r, w1t_r, b1_r,
                          nk).astype(BF)
        z = jnp.concatenate([y1b, y1b * y1b], axis=1)

        @pl.when(pl.program_id(1) == 0)
        def _():
            acc1_r[...] = jnp.zeros_like(acc1_r)

        for i, (s0, span, r0) in enumerate(subs):
            oh = _band_onehot(st_r, en_r, s0, r0)
            part = jnp.dot(oh, z[i * R:(i + 1) * R, :],
                           preferred_element_type=jnp.float32)
            _scatter_acc(acc1_r, s0, span, part, 2 * mid)

    acc1 = pl.pallas_call(
        pass_a,
        grid_spec=pltpu.PrefetchScalarGridSpec(
            num_scalar_prefetch=1,
            grid=(P, nsteps),
            in_specs=[
                relt_spec, row_spec(h_dim), const_spec((1, nk)),
                const_spec((2 * nk, 16 * nk)), const_spec((1, 16 * nk)),
                const_spec((16 * nk + h_dim, mid)), const_spec((1, mid)),
                const_spec((sp, R)), const_spec((sp, R)),
            ],
            out_specs=acc_spec(2 * mid),
        ),
        out_shape=jax.ShapeDtypeStruct((P, sp, 2 * mid), f32),
        compiler_params=pltpu.CompilerParams(
            dimension_semantics=("parallel", "arbitrary"),
            vmem_limit_bytes=40 * 1024 * 1024,
        ),
        name="gravity_stats1",
    )(plan, rel2t, h_state, mass, wst, bs2, w1t, b1r, starts_rep, ends_rep)

    AB = 128

    def _affine_call(acc, gam, bet, d, name):
        def aff_k(acc_r, st_r, en_r, g_r, be_r, af_r):
            sums = acc_r[0] + acc_r[1]
            cnt = (en_r[:, 0:1] - st_r[:, 0:1]).astype(jnp.float32)
            inv_cnt = 1.0 / jnp.maximum(cnt, 1.0)
            mean = sums[:, :d] * inv_cnt
            var = sums[:, d:] * inv_cnt - mean * mean
            inv = jax.lax.rsqrt(var + EPS)
            a = inv * g_r[...]
            c = be_r[...] - mean * a
            af_r[...] = jnp.concatenate([a, c], axis=1).astype(BF)

        return pl.pallas_call(
            aff_k,
            grid=(sp // AB,),
            in_specs=[
                pl.BlockSpec((P, AB, 2 * d), lambda j: (0, j, 0)),
                pl.BlockSpec((AB, R), lambda j: (j, 0)),
                pl.BlockSpec((AB, R), lambda j: (j, 0)),
                pl.BlockSpec((1, d), lambda j: (0, 0)),
                pl.BlockSpec((1, d), lambda j: (0, 0)),
            ],
            out_specs=pl.BlockSpec((AB, 2 * d), lambda j: (j, 0)),
            out_shape=jax.ShapeDtypeStruct((sp, 2 * d), BF),
            compiler_params=pltpu.CompilerParams(
                dimension_semantics=("arbitrary",)),
            name=name,
        )(acc, starts_rep, ends_rep, gam, bet)

    af1 = _affine_call(acc1, g1r, be1r, mid, "gravity_affine1")

    def pass_b(sr, rel_r, h_r, mass_r, wst_r, bs_r, w1t_r, b1_r,
               w2t_r, b2_r, st_r, en_r, af1_r, y2_r, acc2_r):
        subs = _sub(sr, pl.program_id(0), pl.program_id(1))
        y1 = _gravity_y1(rel_r, h_r, mass_r, wst_r, bs_r, w1t_r, b1_r, nk)
        d1 = y1.shape[1]

        ohs = []
        h1_parts = []
        for i, (s0, span, r0) in enumerate(subs):
            oh = _band_onehot(st_r, en_r, s0, r0)
            ohs.append(oh)
            rows = _gather_rows(oh, af1_r[pl.ds(s0, WB), :])
            y1_i = y1[i * R:(i + 1) * R, :]
            h1_parts.append(
                jnp.maximum(y1_i * rows[:, :d1] + rows[:, d1:], 0.0)
                .astype(BF))
        h1 = jnp.concatenate(h1_parts, axis=0)

        y2 = jnp.dot(h1, w2t_r[...], preferred_element_type=jnp.float32)
        y2 = (y2 + b2_r[...]).astype(BF)
        y2_r[...] = y2

        @pl.when(pl.program_id(1) == 0)
        def _():
            acc2_r[...] = jnp.zeros_like(acc2_r)

        for i, (s0, span, r0) in enumerate(subs):
            y2_i = y2[i * R:(i + 1) * R, :]
            z2 = jnp.concatenate([y2_i, y2_i * y2_i], axis=1)
            part2 = jnp.dot(ohs[i], z2, preferred_element_type=jnp.float32)
            _scatter_acc(acc2_r, s0, span, part2, 2 * bot)

    y2_full, acc2 = pl.pallas_call(
        pass_b,
        grid_spec=pltpu.PrefetchScalarGridSpec(
            num_scalar_prefetch=1,
            grid=(P, nsteps),
            in_specs=[
                relt_spec, row_spec(h_dim), const_spec((1, nk)),
                const_spec((2 * nk, 16 * nk)), const_spec((1, 16 * nk)),
                const_spec((16 * nk + h_dim, mid)), const_spec((1, mid)),
                const_spec((mid, bot)), const_spec((1, bot)),
                const_spec((sp, R)), const_spec((sp, R)),
                const_spec((sp, 2 * mid)),
            ],
            out_specs=[row_spec(bot), acc_spec(2 * bot)],
        ),
        out_shape=[
            jax.ShapeDtypeStruct((n, bot), BF),
            jax.ShapeDtypeStruct((P, sp, 2 * bot), f32),
        ],
        compiler_params=pltpu.CompilerParams(
            dimension_semantics=("parallel", "arbitrary"),
            vmem_limit_bytes=52 * 1024 * 1024,
        ),
        name="gravity_mid",
    )(plan, rel2t, h_state, mass, wst, bs2, w1t, b1r, w2t, b2r,
      starts_rep, ends_rep, af1)

    af2 = _affine_call(acc2, g2r, be2r, bot, "gravity_affine2")

    def pass_c(sr, y2_r, st_r, en_r, af2_r, out_r):
        subs = _sub(sr, pl.program_id(0), pl.program_id(1))
        for i, (s0, span, r0) in enumerate(subs):
            oh = _band_onehot(st_r, en_r, s0, r0)
            rows = _gather_rows(oh, af2_r[pl.ds(s0, WB), :])
            y2 = y2_r[i * R:(i + 1) * R, :].astype(jnp.float32)
            d2 = y2.shape[1]
            out_r[i * R:(i + 1) * R, :] = jnp.maximum(
                y2 * rows[:, :d2] + rows[:, d2:], 0.0)

    out = pl.pallas_call(
        pass_c,
        grid_spec=pltpu.PrefetchScalarGridSpec(
            num_scalar_prefetch=1,
            grid=(P, nsteps),
            in_specs=[
                row_spec(bot),
                const_spec((sp, R)), const_spec((sp, R)),
                const_spec((sp, 2 * bot)),
            ],
            out_specs=row_spec(bot),
        ),
        out_shape=jax.ShapeDtypeStruct((n, bot), f32),
        compiler_params=pltpu.CompilerParams(
            dimension_semantics=("parallel", "arbitrary"),
            vmem_limit_bytes=52 * 1024 * 1024,
        ),
        name="gravity_norm2",
    )(plan, y2_full, starts_rep, ends_rep, af2)

    return out


# speedup vs baseline: 1.9303x; 1.0526x over previous
import jax
import jax.numpy as jnp
from jax.experimental import pallas as pl
from jax.experimental.pallas import tpu as pltpu

EPS = 1e-5
R = 128
WB = R + 8
U = 16
P = 2

BF = jnp.bfloat16


def _band_onehot(starts_ref, ends_ref, s0a, r0):
    sb = starts_ref[pl.ds(s0a, WB), :]
    eb = ends_ref[pl.ds(s0a, WB), :]
    row = jax.lax.broadcasted_iota(jnp.int32, (1, R), 1) + r0
    mask = (row >= sb) & (row < eb)
    return jnp.where(mask, 1.0, 0.0).astype(BF)


def _band_affine(acc_band, cnt, gamma, beta, d):
    inv_cnt = 1.0 / jnp.maximum(cnt, 1.0)
    mean = acc_band[:, :d] * inv_cnt
    var = acc_band[:, d:] * inv_cnt - mean * mean
    inv = jax.lax.rsqrt(var + EPS)
    a = inv * gamma
    c = beta - mean * a
    return jnp.concatenate([a, c], axis=1)


def _gather_rows(onehot, band_mat):
    return jax.lax.dot_general(
        onehot, band_mat, (((0,), (0,)), ((), ())),
        preferred_element_type=jnp.float32)


def _gravity_y1(rel_ref, h_ref, mass_ref, wst_ref, bs_ref, w1t_ref, b1_ref, nk):
    rel_t = rel_ref[...]
    rows = []
    for k in range(nk):
        x = rel_t[k:k + 1, :]
        y = rel_t[nk + k:nk + k + 1, :]
        inv_d2 = 1.0 / (x * x + y * y)
        f = mass_ref[0, k] * inv_d2
        rows.append(-x * f)
        rows.append(-y * f)
    rep_t = jnp.concatenate(rows, axis=0)
    emb = jax.lax.dot_general(
        rep_t, wst_ref[...], (((0,), (0,)), ((), ())),
        preferred_element_type=jnp.float32)
    emb = emb + bs_ref[...]
    xcat = jnp.concatenate([emb, h_ref[...]], axis=1).astype(BF)
    y1 = jnp.dot(xcat, w1t_ref[...], preferred_element_type=jnp.float32)
    return y1 + b1_ref[...]


def kernel(h_state, seq_start_end, curr_block_rel, biker_mass, obstacle_mass,
           Ws, bs, W1, b1, g1, be1, W2, b2, g2, be2):
    n, h_dim = h_state.shape
    nk = curr_block_rel.shape[2]
    s = seq_start_end.shape[0]
    mid = W1.shape[0]
    bot = W2.shape[0]
    sp = s + 2 * WB
    nb = n // R
    nsteps = nb // (P * U)
    rb = U * R

    f32 = jnp.float32
    rel2t = curr_block_rel.reshape(n, 2 * nk).T.astype(f32)
    mass = (biker_mass[0] * obstacle_mass).reshape(1, nk).astype(f32)
    wst = Ws.T
    w1t = W1.T.astype(BF)
    w2t = W2.T.astype(BF)
    bs2 = bs.reshape(1, -1)
    b1r = b1.reshape(1, mid)
    g1r = g1.reshape(1, mid)
    be1r = be1.reshape(1, mid)
    b2r = b2.reshape(1, bot)
    g2r = g2.reshape(1, bot)
    be2r = be2.reshape(1, bot)

    starts = seq_start_end[:, 0].astype(jnp.int32)
    ends = seq_start_end[:, 1].astype(jnp.int32)
    padv = jnp.full((sp - s,), n, dtype=jnp.int32)
    starts_rep = jnp.broadcast_to(
        jnp.concatenate([starts, padv])[:, None], (sp, R))
    ends_rep = jnp.broadcast_to(
        jnp.concatenate([ends, padv])[:, None], (sp, R))
    blk0 = jnp.arange(nb, dtype=jnp.int32) * R
    s0a = ((jnp.searchsorted(ends, blk0, side="right").astype(jnp.int32)
            // 8) * 8)
    s1 = jnp.searchsorted(ends, blk0 + (R - 1), side="right").astype(jnp.int32)
    span = s1 - s0a + 1
    plan = jnp.concatenate([s0a, span])

    row_spec = lambda d: pl.BlockSpec(
        (rb, d), lambda p, j, sr: (p * nsteps + j, 0))
    relt_spec = pl.BlockSpec((2 * nk, rb), lambda p, j, sr: (0, p * nsteps + j))
    const_spec = lambda shape: pl.BlockSpec(
        shape, lambda p, j, sr: tuple(0 for _ in shape))
    acc_spec = lambda d: pl.BlockSpec((1, sp, d), lambda p, j, sr: (p, 0, 0))

    nsteps_c = nsteps

    nb_c = nb

    def _sub(sr, pgid, j):
        g0 = (pgid * nsteps_c + j) * U
        return [(pl.multiple_of(sr[g0 + i], 8), sr[nb_c + g0 + i],
                 (g0 + i) * R) for i in range(U)]

    def _scatter_acc(acc_r, s0, span, part, d):
        for c in range(0, WB, 32):
            w = min(32, WB - c)

            @pl.when(c < span)
            def _():
                acc_r[0, pl.ds(s0 + c, w), :] += part[c:c + w, :]

    def pass_a(sr, rel_r, h_r, mass_r, wst_r, bs_r, w1t_r, b1_r,
               st_r, en_r, acc1_r):
        subs = _sub(sr, pl.program_id(0), pl.program_id(1))
        y1b = _gravity_y1(rel_r, h_r, mass_r, wst_r, bs_r, w1t_r, b1_r,
                          nk).astype(BF)
        z = jnp.concatenate([y1b, y1b * y1b], axis=1)

        @pl.when(pl.program_id(1) == 0)
        def _():
            acc1_r[...] = jnp.zeros_like(acc1_r)

        for i, (s0, span, r0) in enumerate(subs):
            oh = _band_onehot(st_r, en_r, s0, r0)
            part = jnp.dot(oh, z[i * R:(i + 1) * R, :],
                           preferred_element_type=jnp.float32)
            _scatter_acc(acc1_r, s0, span, part, 2 * mid)

    acc1 = pl.pallas_call(
        pass_a,
        grid_spec=pltpu.PrefetchScalarGridSpec(
            num_scalar_prefetch=1,
            grid=(P, nsteps),
            in_specs=[
                relt_spec, row_spec(h_dim), const_spec((1, nk)),
                const_spec((2 * nk, 16 * nk)), const_spec((1, 16 * nk)),
                const_spec((16 * nk + h_dim, mid)), const_spec((1, mid)),
                const_spec((sp, R)), const_spec((sp, R)),
            ],
            out_specs=acc_spec(2 * mid),
        ),
        out_shape=jax.ShapeDtypeStruct((P, sp, 2 * mid), f32),
        compiler_params=pltpu.CompilerParams(
            dimension_semantics=("parallel", "arbitrary"),
            vmem_limit_bytes=40 * 1024 * 1024,
        ),
        name="gravity_stats1",
    )(plan, rel2t, h_state, mass, wst, bs2, w1t, b1r, starts_rep, ends_rep)

    AB = 128

    def _affine_call(acc, gam, bet, d, name):
        def aff_k(acc_r, st_r, en_r, g_r, be_r, af_r):
            sums = acc_r[0] + acc_r[1]
            cnt = (en_r[:, 0:1] - st_r[:, 0:1]).astype(jnp.float32)
            inv_cnt = 1.0 / jnp.maximum(cnt, 1.0)
            mean = sums[:, :d] * inv_cnt
            var = sums[:, d:] * inv_cnt - mean * mean
            inv = jax.lax.rsqrt(var + EPS)
            a = inv * g_r[...]
            c = be_r[...] - mean * a
            af_r[...] = jnp.concatenate([a, c], axis=1).astype(BF)

        return pl.pallas_call(
            aff_k,
            grid=(sp // AB,),
            in_specs=[
                pl.BlockSpec((P, AB, 2 * d), lambda j: (0, j, 0)),
                pl.BlockSpec((AB, R), lambda j: (j, 0)),
                pl.BlockSpec((AB, R), lambda j: (j, 0)),
                pl.BlockSpec((1, d), lambda j: (0, 0)),
                pl.BlockSpec((1, d), lambda j: (0, 0)),
            ],
            out_specs=pl.BlockSpec((AB, 2 * d), lambda j: (j, 0)),
            out_shape=jax.ShapeDtypeStruct((sp, 2 * d), BF),
            compiler_params=pltpu.CompilerParams(
                dimension_semantics=("arbitrary",)),
            name=name,
        )(acc, starts_rep, ends_rep, gam, bet)

    af1 = _affine_call(acc1, g1r, be1r, mid, "gravity_affine1")

    def pass_b(sr, rel_r, h_r, mass_r, wst_r, bs_r, w1t_r, b1_r,
               w2t_r, b2_r, st_r, en_r, af1_r, y2_r, acc2_r):
        subs = _sub(sr, pl.program_id(0), pl.program_id(1))
        y1 = _gravity_y1(rel_r, h_r, mass_r, wst_r, bs_r, w1t_r, b1_r, nk)
        d1 = y1.shape[1]

        ohs = []
        h1_parts = []
        for i, (s0, span, r0) in enumerate(subs):
            oh = _band_onehot(st_r, en_r, s0, r0)
            ohs.append(oh)
            rows = _gather_rows(oh, af1_r[pl.ds(s0, WB), :])
            y1_i = y1[i * R:(i + 1) * R, :]
            h1_parts.append(
                jnp.maximum(y1_i * rows[:, :d1] + rows[:, d1:], 0.0)
                .astype(BF))
        h1 = jnp.concatenate(h1_parts, axis=0)

        y2 = jnp.dot(h1, w2t_r[...], preferred_element_type=jnp.float32)
        y2 = (y2 + b2_r[...]).astype(BF)
        y2_r[...] = y2

        @pl.when(pl.program_id(1) == 0)
        def _():
            acc2_r[...] = jnp.zeros_like(acc2_r)

        for i, (s0, span, r0) in enumerate(subs):
            y2_i = y2[i * R:(i + 1) * R, :]
            z2 = jnp.concatenate([y2_i, y2_i * y2_i], axis=1)
            part2 = jnp.dot(ohs[i], z2, preferred_element_type=jnp.float32)
            _scatter_acc(acc2_r, s0, span, part2, 2 * bot)

    y2_full, acc2 = pl.pallas_call(
        pass_b,
        grid_spec=pltpu.PrefetchScalarGridSpec(
            num_scalar_prefetch=1,
            grid=(P, nsteps),
            in_specs=[
                relt_spec, row_spec(h_dim), const_spec((1, nk)),
                const_spec((2 * nk, 16 * nk)), const_spec((1, 16 * nk)),
                const_spec((16 * nk + h_dim, mid)), const_spec((1, mid)),
                const_spec((mid, bot)), const_spec((1, bot)),
                const_spec((sp, R)), const_spec((sp, R)),
                const_spec((sp, 2 * mid)),
            ],
            out_specs=[row_spec(bot), acc_spec(2 * bot)],
        ),
        out_shape=[
            jax.ShapeDtypeStruct((n, bot), BF),
            jax.ShapeDtypeStruct((P, sp, 2 * bot), f32),
        ],
        compiler_params=pltpu.CompilerParams(
            dimension_semantics=("parallel", "arbitrary"),
            vmem_limit_bytes=52 * 1024 * 1024,
        ),
        name="gravity_mid",
    )(plan, rel2t, h_state, mass, wst, bs2, w1t, b1r, w2t, b2r,
      starts_rep, ends_rep, af1)

    af2 = _affine_call(acc2, g2r, be2r, bot, "gravity_affine2")

    def pass_c(sr, y2_r, st_r, en_r, af2_r, out_r):
        subs = _sub(sr, pl.program_id(0), pl.program_id(1))
        for i, (s0, span, r0) in enumerate(subs):
            oh = _band_onehot(st_r, en_r, s0, r0)
            rows = _gather_rows(oh, af2_r[pl.ds(s0, WB), :])
            y2 = y2_r[i * R:(i + 1) * R, :].astype(jnp.float32)
            d2 = y2.shape[1]
            out_r[i * R:(i + 1) * R, :] = jnp.maximum(
                y2 * rows[:, :d2] + rows[:, d2:], 0.0)

    out = pl.pallas_call(
        pass_c,
        grid_spec=pltpu.PrefetchScalarGridSpec(
            num_scalar_prefetch=1,
            grid=(P, nsteps),
            in_specs=[
                row_spec(bot),
                const_spec((sp, R)), const_spec((sp, R)),
                const_spec((sp, 2 * bot)),
            ],
            out_specs=row_spec(bot),
        ),
        out_shape=jax.ShapeDtypeStruct((n, bot), f32),
        compiler_params=pltpu.CompilerParams(
            dimension_semantics=("parallel", "arbitrary"),
            vmem_limit_bytes=52 * 1024 * 1024,
        ),
        name="gravity_norm2",
    )(plan, y2_full, starts_rep, ends_rep, af2)

    return out
